# baseline (speedup 1.0000x reference)
import functools

import jax
import jax.numpy as jnp
import numpy as np
from jax import lax
from jax.experimental import pallas as pl
from jax.experimental.pallas import tpu as pltpu
from jax.experimental.pallas import tpu_sc as plsc

_OFFSETS = np.array([0, 6, 12, 15, 25, 32, 40, 42, 46, 53, 53], dtype=np.int64)
_PADDING_IDX = 61
_NUM_BAGS = 10
_DIM = 5
_LANES = 16
_N_IDX = 53
_N_ROWS = 100
_NSLOTS = _NUM_BAGS * _DIM
_NCHUNKS = 4
_INVALID_POS = 63


def _build_tables():
    lens = (_OFFSETS[1:] - _OFFSETS[:-1]).astype(np.int32)
    jmax, pos, dvec = [], [], []
    for r in range(_NCHUNKS):
        slots = np.arange(r * _LANES, (r + 1) * _LANES)
        bags = slots // _DIM
        valid = slots < _NSLOTS
        dvec.append(np.where(valid, slots % _DIM, 0).astype(np.int32))
        jm = int(max([lens[b] for b, v in zip(bags, valid) if v], default=0))
        rows = []
        for j in range(jm):
            p = np.full((_LANES,), _INVALID_POS, np.int32)
            for l in range(_LANES):
                if valid[l] and j < lens[bags[l]]:
                    p[l] = int(_OFFSETS[bags[l]]) + j
            rows.append(p)
        jmax.append(jm)
        pos.append(rows)
    return jmax, pos, dvec


_JMAX, _POS, _DVEC = _build_tables()
_LENS = (_OFFSETS[1:] - _OFFSETS[:-1]).astype(int).tolist()

_mesh = plsc.VectorSubcoreMesh(
    core_axis_name="c", subcore_axis_name="s", num_cores=1, num_subcores=1
)


@functools.partial(
    pl.kernel,
    out_type=jax.ShapeDtypeStruct((_NSLOTS,), jnp.float32),
    mesh=_mesh,
    scratch_types=[
        pltpu.VMEM((_N_IDX,), jnp.int32),
        pltpu.VMEM((_N_ROWS, _DIM), jnp.float32),
        pltpu.VMEM((_LANES,), jnp.int32),
        pltpu.VMEM((_NCHUNKS * _LANES,), jnp.float32),
        pltpu.SemaphoreType.DMA,
        pltpu.SemaphoreType.DMA,
        pltpu.SemaphoreType.DMA,
    ],
    compiler_params=pltpu.CompilerParams(
        needs_layout_passes=False,
        disable_bounds_checks=True,
        disable_semaphore_checks=True,
    ),
)
def _bag_mean_sc(inp_hbm, w_hbm, offs_hbm, out_hbm, inp_v, w_v, offs_v, out_v,
                 sem1, sem2, sem3):
    cp1 = pltpu.async_copy(inp_hbm, inp_v, sem1)
    cp2 = pltpu.async_copy(w_hbm, w_v, sem2)
    cp3 = pltpu.async_copy(offs_hbm, offs_v, sem3)
    cp1.wait()
    cp2.wait()
    cp3.wait()
    lane = lax.broadcasted_iota(jnp.int32, (_LANES,), 0)
    for r in range(_NCHUNKS):
        slot = lane + r * _LANES
        bag = jnp.minimum((slot * 13) >> 6, _NUM_BAGS)
        dvec = slot - bag * _DIM
        off_vec = plsc.load_gather(offs_v, [bag])
        len_vec = plsc.load_gather(offs_v, [jnp.minimum(bag + 1, _NUM_BAGS)]) - off_vec
        acc = jnp.zeros((_LANES,), jnp.float32)
        cnt = jnp.zeros((_LANES,), jnp.float32)
        for j in range(_JMAX[r]):
            posv = jnp.minimum(off_vec + j, _N_IDX - 1)
            idx = plsc.load_gather(inp_v, [posv])
            mf = jnp.where(
                jnp.logical_and(len_vec > j, idx != _PADDING_IDX), 1.0, 0.0
            ).astype(jnp.float32)
            idxc = jnp.minimum(jnp.maximum(idx, 0), _N_ROWS - 1)
            vals = plsc.load_gather(w_v, [idxc, dvec])
            acc = acc + vals * mf
            cnt = cnt + mf
        out_v[pl.ds(r * _LANES, _LANES)] = acc / jnp.maximum(cnt, 1.0)
    pltpu.sync_copy(out_v.at[pl.ds(0, _NSLOTS)], out_hbm)


_OFFS_PADDED = np.concatenate(
    [_OFFSETS.astype(np.int32), np.zeros((_LANES - len(_OFFSETS),), np.int32)]
)


def kernel(input, weight):
    out = _bag_mean_sc(input, weight, jnp.asarray(_OFFS_PADDED))
    return out.reshape(_NUM_BAGS, _DIM)


# speedup vs baseline: 1.0214x; 1.0214x over previous
import functools

import jax
import jax.numpy as jnp
import numpy as np
from jax import lax
from jax.experimental import pallas as pl
from jax.experimental.pallas import tpu as pltpu
from jax.experimental.pallas import tpu_sc as plsc

_OFFSETS = np.array([0, 6, 12, 15, 25, 32, 40, 42, 46, 53, 53], dtype=np.int64)
_PADDING_IDX = 61
_NUM_BAGS = 10
_DIM = 5
_LANES = 16
_N_IDX = 53
_N_ROWS = 100
_NSLOTS = _NUM_BAGS * _DIM
_NCHUNKS = 4
_INVALID_POS = 63


def _build_tables():
    lens = (_OFFSETS[1:] - _OFFSETS[:-1]).astype(np.int32)
    jmax, pos, dvec = [], [], []
    for r in range(_NCHUNKS):
        slots = np.arange(r * _LANES, (r + 1) * _LANES)
        bags = slots // _DIM
        valid = slots < _NSLOTS
        dvec.append(np.where(valid, slots % _DIM, 0).astype(np.int32))
        jm = int(max([lens[b] for b, v in zip(bags, valid) if v], default=0))
        rows = []
        for j in range(jm):
            p = np.full((_LANES,), _INVALID_POS, np.int32)
            for l in range(_LANES):
                if valid[l] and j < lens[bags[l]]:
                    p[l] = int(_OFFSETS[bags[l]]) + j
            rows.append(p)
        jmax.append(jm)
        pos.append(rows)
    return jmax, pos, dvec


_JMAX, _POS, _DVEC = _build_tables()
_LENS = (_OFFSETS[1:] - _OFFSETS[:-1]).astype(int).tolist()

_mesh = plsc.VectorSubcoreMesh(
    core_axis_name="c", subcore_axis_name="s", num_cores=1, num_subcores=1
)


@functools.partial(
    pl.kernel,
    out_type=jax.ShapeDtypeStruct((_NSLOTS,), jnp.float32),
    mesh=_mesh,
    scratch_types=[
        pltpu.VMEM((_N_IDX,), jnp.int32),
        pltpu.VMEM((_N_ROWS, _DIM), jnp.float32),
        pltpu.VMEM((_LANES,), jnp.int32),
        pltpu.VMEM((_NCHUNKS * _LANES,), jnp.float32),
        pltpu.SemaphoreType.DMA,
        pltpu.SemaphoreType.DMA,
        pltpu.SemaphoreType.DMA,
    ],
    compiler_params=pltpu.CompilerParams(
        needs_layout_passes=False,
        disable_bounds_checks=True,
        disable_semaphore_checks=True,
    ),
)
def _bag_mean_sc(inp_hbm, w_hbm, offs_hbm, out_hbm, inp_v, w_v, offs_v, out_v,
                 sem1, sem2, sem3):
    cp1 = pltpu.async_copy(inp_hbm, inp_v, sem1)
    cp2 = pltpu.async_copy(w_hbm, w_v, sem2)
    cp3 = pltpu.async_copy(offs_hbm, offs_v, sem3)
    cp1.wait()
    cp2.wait()
    cp3.wait()
    lane = lax.broadcasted_iota(jnp.int32, (_LANES,), 0)
    maxlen = max(_JMAX)

    def chunk_body(r, _):
        slot = lane + r * _LANES
        bag = jnp.minimum((slot * 13) >> 6, _NUM_BAGS)
        dvec = jnp.minimum(slot - bag * _DIM, _DIM - 1)
        off_vec = plsc.load_gather(offs_v, [bag])
        nxt_vec = plsc.load_gather(offs_v, [jnp.minimum(bag + 1, _NUM_BAGS)])
        len_vec = nxt_vec - off_vec

        def j_body(j, carry):
            acc, cnt = carry
            posv = jnp.minimum(off_vec + j, _N_IDX - 1)
            idx = plsc.load_gather(inp_v, [posv])
            mf = jnp.where(
                jnp.logical_and(len_vec > j, idx != _PADDING_IDX), 1.0, 0.0
            ).astype(jnp.float32)
            idxc = jnp.minimum(jnp.maximum(idx, 0), _N_ROWS - 1)
            vals = plsc.load_gather(w_v, [idxc, dvec])
            return acc + vals * mf, cnt + mf

        acc, cnt = lax.fori_loop(
            0,
            maxlen,
            j_body,
            (jnp.zeros((_LANES,), jnp.float32), jnp.zeros((_LANES,), jnp.float32)),
        )
        out_v[pl.ds(r * _LANES, _LANES)] = acc / jnp.maximum(cnt, 1.0)
        return ()

    lax.fori_loop(0, _NCHUNKS, chunk_body, ())
    pltpu.sync_copy(out_v.at[pl.ds(0, _NSLOTS)], out_hbm)


_OFFS_PADDED = np.concatenate(
    [_OFFSETS.astype(np.int32), np.zeros((_LANES - len(_OFFSETS),), np.int32)]
)


def kernel(input, weight):
    out = _bag_mean_sc(input, weight, jnp.asarray(_OFFS_PADDED))
    return out.reshape(_NUM_BAGS, _DIM)
